# Initial kernel scaffold; baseline (speedup 1.0000x reference)
import jax
import jax.numpy as jnp
from jax.experimental import pallas as pl


def kernel(causal, gamma1, beta1, W1, b1, gamma2, beta2, W2, b2):
    raise NotImplementedError("write your pallas kernel here")


# speedup vs baseline: 1.0560x; 1.0560x over previous
import functools

import jax
import jax.numpy as jnp
from jax import lax
from jax.experimental import pallas as pl
from jax.experimental.pallas import tpu as pltpu

_EPS = 1e-5


def _pick_block(n):
    for blk in (12800, 10000, 8192, 6400, 5000, 4096, 4000, 2048, 2000, 1024, 1000):
        if n % blk == 0:
            return blk
    return n


def _mlp_kernel(x_ref, W1_ref, b1_ref, g1_ref, be1_ref, W2_ref, b2_ref,
                g2_ref, be2_ref, out_ref, s1_ref, q1_ref, s2_ref, q2_ref,
                *, nb, inv_n):
    t = pl.program_id(0)
    phase = t // nb

    @pl.when(t == 0)
    def _init1():
        s1_ref[...] = jnp.zeros_like(s1_ref)
        q1_ref[...] = jnp.zeros_like(q1_ref)

    @pl.when(t == nb)
    def _init2():
        s2_ref[...] = jnp.zeros_like(s2_ref)
        q2_ref[...] = jnp.zeros_like(q2_ref)

    def bn1_affine():
        m1 = s1_ref[...] * inv_n
        v1 = q1_ref[...] * inv_n - m1 * m1
        a1 = g1_ref[...] * lax.rsqrt(v1 + _EPS)
        c1 = be1_ref[...] - m1 * a1
        return a1, c1

    def hidden():
        a1, c1 = bn1_affine()
        xs = x_ref[...] * a1 + c1
        z = lax.dot_general(xs, W1_ref[...], (((1,), (1,)), ((), ())),
                            preferred_element_type=jnp.float32)
        return jnp.maximum(z + b1_ref[...], 0.0)

    @pl.when(phase == 0)
    def _p0():
        xb = x_ref[...]
        s1_ref[...] += jnp.sum(xb, axis=0, keepdims=True)
        q1_ref[...] += jnp.sum(xb * xb, axis=0, keepdims=True)

    @pl.when(phase == 1)
    def _p1():
        h = hidden()
        s2_ref[...] += jnp.sum(h, axis=0, keepdims=True)
        q2_ref[...] += jnp.sum(h * h, axis=0, keepdims=True)

    @pl.when(phase == 2)
    def _p2():
        m2 = s2_ref[...] * inv_n
        v2 = q2_ref[...] * inv_n - m2 * m2
        a2 = g2_ref[...] * lax.rsqrt(v2 + _EPS)
        c2 = be2_ref[...] - m2 * a2
        hs = hidden() * a2 + c2
        out = lax.dot_general(hs, W2_ref[...], (((1,), (1,)), ((), ())),
                              preferred_element_type=jnp.float32)
        out_ref[...] = out + b2_ref[...]


def kernel(causal, gamma1, beta1, W1, b1, gamma2, beta2, W2, b2):
    n, d = causal.shape
    d_out = W2.shape[0]
    blk = _pick_block(n)
    nb = n // blk

    row = lambda v: v.reshape(1, -1)

    def full(shape):
        return pl.BlockSpec(shape, lambda t: (0,) * len(shape))

    x_spec = pl.BlockSpec((blk, d), lambda t: (lax.rem(t, nb), 0))
    out_spec = pl.BlockSpec(
        (blk, d_out),
        lambda t: (jnp.where(t >= 2 * nb, lax.rem(t, nb), 0), 0))

    fn = pl.pallas_call(
        functools.partial(_mlp_kernel, nb=nb, inv_n=1.0 / n),
        grid=(3 * nb,),
        in_specs=[
            x_spec,
            full((d, d)),
            full((1, d)),
            full((1, d)),
            full((1, d)),
            full((d_out, d)),
            full((1, d_out)),
            full((1, d)),
            full((1, d)),
        ],
        out_specs=out_spec,
        out_shape=jax.ShapeDtypeStruct((n, d_out), jnp.float32),
        scratch_shapes=[pltpu.VMEM((1, d), jnp.float32)] * 4,
        compiler_params=pltpu.CompilerParams(
            dimension_semantics=("arbitrary",)),
    )
    return fn(causal, W1, row(b1), row(gamma1), row(beta1),
              W2, row(b2), row(gamma2), row(beta2))
